# Initial kernel scaffold; baseline (speedup 1.0000x reference)
import jax
import jax.numpy as jnp
from jax.experimental import pallas as pl


def kernel(boxes, scores):
    raise NotImplementedError("write your pallas kernel here")


# speedup vs baseline: 374.6341x; 374.6341x over previous
import functools

import jax
import jax.numpy as jnp
from jax.experimental import pallas as pl
from jax.experimental.pallas import tpu as pltpu

_NMS_THR = 0.3
_SCORE_THR = 0.1
_MAX_PRED = 100
_R = 32
_L = 128
_M = _R * _L


def _nms_kernel(x1_ref, y1_ref, x2_ref, y2_ref, s_ref,
                bo_ref, so_ref, co_ref):
    x1 = x1_ref[0]
    y1 = y1_ref[0]
    x2 = x2_ref[0]
    y2 = y2_ref[0]
    sc = s_ref[0]

    valid = (sc > _SCORE_THR) & (x2 > x1) & (y2 > y1)
    neg_inf = jnp.float32(-jnp.inf)
    s0 = jnp.where(valid, sc, neg_inf)
    area = jnp.maximum(x2 - x1, 0.0) * jnp.maximum(y2 - y1, 0.0)

    flat = (jax.lax.broadcasted_iota(jnp.int32, (_R, _L), 0) * _L
            + jax.lax.broadcasted_iota(jnp.int32, (_R, _L), 1))

    bo_ref[0] = jnp.zeros((_MAX_PRED, 4), jnp.float32)
    so_ref[0] = jnp.zeros((_MAX_PRED, 1), jnp.float32)
    co_ref[0] = jnp.full((_MAX_PRED, 1), -1, jnp.int32)

    def body(k, s):
        m = jnp.max(s)

        def picked(s):
            pick = jnp.min(jnp.where(s == m, flat, _M))
            onehot = flat == pick
            bx1 = jnp.sum(jnp.where(onehot, x1, 0.0))
            by1 = jnp.sum(jnp.where(onehot, y1, 0.0))
            bx2 = jnp.sum(jnp.where(onehot, x2, 0.0))
            by2 = jnp.sum(jnp.where(onehot, y2, 0.0))
            barea = jnp.sum(jnp.where(onehot, area, 0.0))

            iw = jnp.maximum(jnp.minimum(x2, bx2) - jnp.maximum(x1, bx1), 0.0)
            ih = jnp.maximum(jnp.minimum(y2, by2) - jnp.maximum(y1, by1), 0.0)
            inter = iw * ih
            union = jnp.maximum(area + barea - inter, 1e-9)
            kill = (inter / union > _NMS_THR) | onehot

            bo_ref[0, pl.ds(k, 1), 0:1] = bx1.reshape(1, 1)
            bo_ref[0, pl.ds(k, 1), 1:2] = by1.reshape(1, 1)
            bo_ref[0, pl.ds(k, 1), 2:3] = bx2.reshape(1, 1)
            bo_ref[0, pl.ds(k, 1), 3:4] = by2.reshape(1, 1)
            so_ref[0, pl.ds(k, 1), :] = m.reshape(1, 1)
            co_ref[0, pl.ds(k, 1), :] = (pick & 1).reshape(1, 1)
            return jnp.where(kill, neg_inf, s)

        return jax.lax.cond(m > neg_inf, picked, lambda s: s, s)

    jax.lax.fori_loop(0, _MAX_PRED, body, s0)


@jax.jit
def kernel(boxes, scores):
    B, N, C = scores.shape[0], scores.shape[1], scores.shape[2] - 1
    M = N * C
    pad = _M - M

    bf = boxes.reshape(B, M, 4)
    x1 = jnp.pad(bf[:, :, 0], ((0, 0), (0, pad))).reshape(B, _R, _L)
    y1 = jnp.pad(bf[:, :, 1], ((0, 0), (0, pad))).reshape(B, _R, _L)
    x2 = jnp.pad(bf[:, :, 2], ((0, 0), (0, pad))).reshape(B, _R, _L)
    y2 = jnp.pad(bf[:, :, 3], ((0, 0), (0, pad))).reshape(B, _R, _L)
    sf = jnp.pad(scores[:, :, 1:].reshape(B, M), ((0, 0), (0, pad)),
                 constant_values=-1.0).reshape(B, _R, _L)

    in_spec = pl.BlockSpec((1, _R, _L), lambda b: (b, 0, 0))
    bo, so, co = pl.pallas_call(
        _nms_kernel,
        grid=(B,),
        in_specs=[in_spec] * 5,
        out_specs=[
            pl.BlockSpec((1, _MAX_PRED, 4), lambda b: (b, 0, 0)),
            pl.BlockSpec((1, _MAX_PRED, 1), lambda b: (b, 0, 0)),
            pl.BlockSpec((1, _MAX_PRED, 1), lambda b: (b, 0, 0)),
        ],
        out_shape=[
            jax.ShapeDtypeStruct((B, _MAX_PRED, 4), jnp.float32),
            jax.ShapeDtypeStruct((B, _MAX_PRED, 1), jnp.float32),
            jax.ShapeDtypeStruct((B, _MAX_PRED, 1), jnp.int32),
        ],
    )(x1, y1, x2, y2, sf)

    return bo, so[:, :, 0], co[:, :, 0]


# speedup vs baseline: 375.5466x; 1.0024x over previous
import functools

import jax
import jax.numpy as jnp
from jax.experimental import pallas as pl
from jax.experimental.pallas import tpu as pltpu

_NMS_THR = 0.3
_SCORE_THR = 0.1
_MAX_PRED = 100
_R = 32
_L = 128
_M = _R * _L


def _nms_kernel(x1_ref, y1_ref, x2_ref, y2_ref, s_ref,
                bo_ref, so_ref, co_ref, area_scr):
    neg_inf = jnp.float32(-jnp.inf)
    flat = (jax.lax.broadcasted_iota(jnp.int32, (_R, _L), 0) * _L
            + jax.lax.broadcasted_iota(jnp.int32, (_R, _L), 1))

    B = s_ref.shape[0]
    s0 = []
    for b in range(B):
        sc = s_ref[b]
        x1, y1, x2, y2 = x1_ref[b], y1_ref[b], x2_ref[b], y2_ref[b]
        valid = (sc > _SCORE_THR) & (x2 > x1) & (y2 > y1)
        s0.append(jnp.where(valid, sc, neg_inf))
        area_scr[b] = jnp.maximum(x2 - x1, 0.0) * jnp.maximum(y2 - y1, 0.0)

    bo_ref[...] = jnp.zeros((B, _MAX_PRED, 4), jnp.float32)
    so_ref[...] = jnp.zeros((B, _MAX_PRED, 1), jnp.float32)
    co_ref[...] = jnp.full((B, _MAX_PRED, 1), -1, jnp.int32)

    def pick_one(b, k, s):
        m = jnp.max(s)

        def picked(s):
            x1, y1, x2, y2 = x1_ref[b], y1_ref[b], x2_ref[b], y2_ref[b]
            area = area_scr[b]
            pick = jnp.min(jnp.where(s == m, flat, _M))
            onehot = flat == pick
            bx1 = jnp.sum(jnp.where(onehot, x1, 0.0))
            by1 = jnp.sum(jnp.where(onehot, y1, 0.0))
            bx2 = jnp.sum(jnp.where(onehot, x2, 0.0))
            by2 = jnp.sum(jnp.where(onehot, y2, 0.0))
            barea = jnp.sum(jnp.where(onehot, area, 0.0))

            iw = jnp.maximum(jnp.minimum(x2, bx2) - jnp.maximum(x1, bx1), 0.0)
            ih = jnp.maximum(jnp.minimum(y2, by2) - jnp.maximum(y1, by1), 0.0)
            inter = iw * ih
            union = jnp.maximum(area + barea - inter, 1e-9)
            kill = (inter / union > _NMS_THR) | onehot

            bo_ref[b, pl.ds(k, 1), 0:1] = bx1.reshape(1, 1)
            bo_ref[b, pl.ds(k, 1), 1:2] = by1.reshape(1, 1)
            bo_ref[b, pl.ds(k, 1), 2:3] = bx2.reshape(1, 1)
            bo_ref[b, pl.ds(k, 1), 3:4] = by2.reshape(1, 1)
            so_ref[b, pl.ds(k, 1), :] = m.reshape(1, 1)
            co_ref[b, pl.ds(k, 1), :] = (pick & 1).reshape(1, 1)
            return jnp.where(kill, neg_inf, s)

        return jax.lax.cond(m > neg_inf, picked, lambda s: s, s)

    def body(k, ss):
        return tuple(pick_one(b, k, ss[b]) for b in range(B))

    jax.lax.fori_loop(0, _MAX_PRED, body, tuple(s0))


@jax.jit
def kernel(boxes, scores):
    B, N, C = scores.shape[0], scores.shape[1], scores.shape[2] - 1
    M = N * C
    pad = _M - M

    bf = boxes.reshape(B, M, 4)
    x1 = jnp.pad(bf[:, :, 0], ((0, 0), (0, pad))).reshape(B, _R, _L)
    y1 = jnp.pad(bf[:, :, 1], ((0, 0), (0, pad))).reshape(B, _R, _L)
    x2 = jnp.pad(bf[:, :, 2], ((0, 0), (0, pad))).reshape(B, _R, _L)
    y2 = jnp.pad(bf[:, :, 3], ((0, 0), (0, pad))).reshape(B, _R, _L)
    sf = jnp.pad(scores[:, :, 1:].reshape(B, M), ((0, 0), (0, pad)),
                 constant_values=-1.0).reshape(B, _R, _L)

    bo, so, co = pl.pallas_call(
        _nms_kernel,
        out_shape=[
            jax.ShapeDtypeStruct((B, _MAX_PRED, 4), jnp.float32),
            jax.ShapeDtypeStruct((B, _MAX_PRED, 1), jnp.float32),
            jax.ShapeDtypeStruct((B, _MAX_PRED, 1), jnp.int32),
        ],
        scratch_shapes=[pltpu.VMEM((B, _R, _L), jnp.float32)],
    )(x1, y1, x2, y2, sf)

    return bo, so[:, :, 0], co[:, :, 0]


# speedup vs baseline: 505.6227x; 1.3464x over previous
import functools

import jax
import jax.numpy as jnp
from jax.experimental import pallas as pl
from jax.experimental.pallas import tpu as pltpu

_NMS_THR = 0.3
_SCORE_THR = 0.1
_MAX_PRED = 100
_R = 32
_L = 128
_M = _R * _L


def _nms_kernel(x1_ref, y1_ref, x2_ref, y2_ref, s_ref,
                bo_ref, so_ref, co_ref, area_scr):
    neg_inf = jnp.float32(-jnp.inf)
    flat = (jax.lax.broadcasted_iota(jnp.int32, (_R, _L), 0) * _L
            + jax.lax.broadcasted_iota(jnp.int32, (_R, _L), 1))
    lane5 = jax.lax.broadcasted_iota(jnp.int32, (5, _L), 1)

    B = s_ref.shape[0]
    s0 = []
    for b in range(B):
        sc = s_ref[b]
        x1, y1, x2, y2 = x1_ref[b], y1_ref[b], x2_ref[b], y2_ref[b]
        valid = (sc > _SCORE_THR) & (x2 > x1) & (y2 > y1)
        s0.append(jnp.where(valid, sc, neg_inf))
        area_scr[b] = jnp.maximum(x2 - x1, 0.0) * jnp.maximum(y2 - y1, 0.0)

    bo_ref[...] = jnp.zeros(bo_ref.shape, jnp.float32)
    so_ref[...] = jnp.zeros(so_ref.shape, jnp.float32)
    co_ref[...] = jnp.full(co_ref.shape, -1, jnp.int32)

    def pick_one(b, k, s):
        m = jnp.max(s)
        alive = m > neg_inf
        pick = jnp.min(jnp.where(s == m, flat, _M))
        r = jnp.minimum(pick >> 7, _R - 1)
        c = pick & (_L - 1)

        rows = jnp.concatenate(
            [x1_ref[b, pl.ds(r, 1), :],
             y1_ref[b, pl.ds(r, 1), :],
             x2_ref[b, pl.ds(r, 1), :],
             y2_ref[b, pl.ds(r, 1), :],
             area_scr[b, pl.ds(r, 1), :]], axis=0)
        col = jnp.sum(jnp.where(lane5 == c, rows, 0.0), axis=1, keepdims=True)
        bx1, by1 = col[0:1, :], col[1:2, :]
        bx2, by2 = col[2:3, :], col[3:4, :]
        barea = col[4:5, :]

        x1, y1, x2, y2 = x1_ref[b], y1_ref[b], x2_ref[b], y2_ref[b]
        area = area_scr[b]
        iw = jnp.maximum(jnp.minimum(x2, bx2) - jnp.maximum(x1, bx1), 0.0)
        ih = jnp.maximum(jnp.minimum(y2, by2) - jnp.maximum(y1, by1), 0.0)
        inter = iw * ih
        union = jnp.maximum(area + barea - inter, 1e-9)
        kill = (inter / union > _NMS_THR) | (flat == pick)

        rowk = jnp.where(alive, k, _MAX_PRED)
        bo_ref[b, pl.ds(rowk, 1), 0:1] = bx1
        bo_ref[b, pl.ds(rowk, 1), 1:2] = by1
        bo_ref[b, pl.ds(rowk, 1), 2:3] = bx2
        bo_ref[b, pl.ds(rowk, 1), 3:4] = by2
        so_ref[b, pl.ds(rowk, 1), :] = m.reshape(1, 1)
        co_ref[b, pl.ds(rowk, 1), :] = (pick & 1).reshape(1, 1)
        return jnp.where(kill, neg_inf, s)

    def body(k, ss):
        return tuple(pick_one(b, k, ss[b]) for b in range(B))

    jax.lax.fori_loop(0, _MAX_PRED, body, tuple(s0))


@jax.jit
def kernel(boxes, scores):
    B, N, C = scores.shape[0], scores.shape[1], scores.shape[2] - 1
    M = N * C
    pad = _M - M

    bf = boxes.reshape(B, M, 4)
    x1 = jnp.pad(bf[:, :, 0], ((0, 0), (0, pad))).reshape(B, _R, _L)
    y1 = jnp.pad(bf[:, :, 1], ((0, 0), (0, pad))).reshape(B, _R, _L)
    x2 = jnp.pad(bf[:, :, 2], ((0, 0), (0, pad))).reshape(B, _R, _L)
    y2 = jnp.pad(bf[:, :, 3], ((0, 0), (0, pad))).reshape(B, _R, _L)
    sf = jnp.pad(scores[:, :, 1:].reshape(B, M), ((0, 0), (0, pad)),
                 constant_values=-1.0).reshape(B, _R, _L)

    bo, so, co = pl.pallas_call(
        _nms_kernel,
        out_shape=[
            jax.ShapeDtypeStruct((B, _MAX_PRED + 1, 4), jnp.float32),
            jax.ShapeDtypeStruct((B, _MAX_PRED + 1, 1), jnp.float32),
            jax.ShapeDtypeStruct((B, _MAX_PRED + 1, 1), jnp.int32),
        ],
        scratch_shapes=[pltpu.VMEM((B, _R, _L), jnp.float32)],
    )(x1, y1, x2, y2, sf)

    return bo[:, :_MAX_PRED], so[:, :_MAX_PRED, 0], co[:, :_MAX_PRED, 0]


# speedup vs baseline: 773.0001x; 1.5288x over previous
import functools

import jax
import jax.numpy as jnp
from jax.experimental import pallas as pl
from jax.experimental.pallas import tpu as pltpu

_NMS_THR = 0.3
_SCORE_THR = 0.1
_MAX_PRED = 100
_R = 32
_L = 128
_M = _R * _L
_OFF = 0x3DCCCCCD
_SIGN = -2147483648


def _nms_kernel(x1_ref, y1_ref, x2_ref, y2_ref, s_ref,
                bo_ref, so_ref, co_ref):
    neg_inf = jnp.float32(-jnp.inf)
    row = jax.lax.broadcasted_iota(jnp.int32, (_R, _L), 0)
    lane1 = jax.lax.broadcasted_iota(jnp.int32, (1, _L), 1)
    rowpar = (row & 1).astype(jnp.float32)

    B = s_ref.shape[0]
    planes = []
    s0 = []
    for b in range(B):
        sc = s_ref[b]
        x1, y1, x2, y2 = x1_ref[b], y1_ref[b], x2_ref[b], y2_ref[b]
        valid = (sc > _SCORE_THR) & (x2 > x1) & (y2 > y1)
        s0.append(jnp.where(valid, sc, neg_inf))
        area = jnp.maximum(x2 - x1, 0.0) * jnp.maximum(y2 - y1, 0.0)
        planes.append((x1, y1, x2, y2, area))

    bo_ref[...] = jnp.zeros(bo_ref.shape, jnp.float32)
    so_ref[...] = jnp.zeros(so_ref.shape, jnp.float32)
    co_ref[...] = jnp.full(co_ref.shape, -1, jnp.int32)

    def pick_one(b, k, s):
        x1, y1, x2, y2, area = planes[b]
        colmax = jnp.max(s, axis=0, keepdims=True)
        bits = jax.lax.bitcast_convert_type(colmax, jnp.int32)
        key = jnp.where(colmax > neg_inf,
                        ((bits - _OFF) << 7) | (127 - lane1),
                        0)
        kmax = jnp.max(key ^ _SIGN, axis=1, keepdims=True) ^ _SIGN

        c = 127 - (kmax & 127)
        mbits = jax.lax.shift_right_logical(kmax, 7) + _OFF
        m = jax.lax.bitcast_convert_type(mbits, jnp.float32)

        mask2 = (s == m) & (lane1 == c)
        rowsel = jnp.where(mask2, row, _R)
        rmin = jnp.min(rowsel, axis=0, keepdims=True)
        onehot = mask2 & (row == rmin)

        def ex(p):
            t = jnp.sum(jnp.where(onehot, p, 0.0), axis=0, keepdims=True)
            return jnp.sum(t, axis=1, keepdims=True)

        bx1, by1, bx2, by2 = ex(x1), ex(y1), ex(x2), ex(y2)
        barea = ex(area)
        bcls = ex(rowpar)

        iw = jnp.maximum(jnp.minimum(x2, bx2) - jnp.maximum(x1, bx1), 0.0)
        ih = jnp.maximum(jnp.minimum(y2, by2) - jnp.maximum(y1, by1), 0.0)
        inter = iw * ih
        union = jnp.maximum(area + barea - inter, 1e-9)
        kill = (inter / union > _NMS_THR) | onehot

        rowk = jnp.where(kmax[0, 0] != 0, k, _MAX_PRED)
        bo_ref[b, pl.ds(rowk, 1), 0:1] = bx1
        bo_ref[b, pl.ds(rowk, 1), 1:2] = by1
        bo_ref[b, pl.ds(rowk, 1), 2:3] = bx2
        bo_ref[b, pl.ds(rowk, 1), 3:4] = by2
        so_ref[b, pl.ds(rowk, 1), :] = m
        co_ref[b, pl.ds(rowk, 1), :] = bcls.astype(jnp.int32)
        return jnp.where(kill, neg_inf, s)

    def body(k, ss):
        return tuple(pick_one(b, k, ss[b]) for b in range(B))

    jax.lax.fori_loop(0, _MAX_PRED, body, tuple(s0))


def _to_plane(a, pad):
    B = a.shape[0]
    return jnp.pad(a, ((0, 0), (0, pad))).reshape(B, _L, _R).transpose(0, 2, 1)


@jax.jit
def kernel(boxes, scores):
    B, N, C = scores.shape[0], scores.shape[1], scores.shape[2] - 1
    M = N * C
    pad = _M - M

    bf = boxes.reshape(B, M, 4)
    x1 = _to_plane(bf[:, :, 0], pad)
    y1 = _to_plane(bf[:, :, 1], pad)
    x2 = _to_plane(bf[:, :, 2], pad)
    y2 = _to_plane(bf[:, :, 3], pad)
    sf = jnp.pad(scores[:, :, 1:].reshape(B, M), ((0, 0), (0, pad)),
                 constant_values=-1.0).reshape(B, _L, _R).transpose(0, 2, 1)

    bo, so, co = pl.pallas_call(
        _nms_kernel,
        out_shape=[
            jax.ShapeDtypeStruct((B, _MAX_PRED + 1, 4), jnp.float32),
            jax.ShapeDtypeStruct((B, _MAX_PRED + 1, 1), jnp.float32),
            jax.ShapeDtypeStruct((B, _MAX_PRED + 1, 1), jnp.int32),
        ],
    )(x1, y1, x2, y2, sf)

    return bo[:, :_MAX_PRED], so[:, :_MAX_PRED, 0], co[:, :_MAX_PRED, 0]


# speedup vs baseline: 872.8455x; 1.1292x over previous
import functools

import jax
import jax.numpy as jnp
from jax.experimental import pallas as pl
from jax.experimental.pallas import tpu as pltpu

_NMS_THR = 0.3
_SCORE_THR = 0.1
_MAX_PRED = 100
_R = 32
_L = 128
_M = _R * _L
_OFF = 0x3DCCCCCD
_SIGN = -2147483648


def _nms_kernel(x1_ref, y1_ref, x2_ref, y2_ref, s_ref,
                bo_ref, so_ref, co_ref):
    neg_inf = jnp.float32(-jnp.inf)
    row = jax.lax.broadcasted_iota(jnp.int32, (_R, _L), 0)
    lane1 = jax.lax.broadcasted_iota(jnp.int32, (1, _L), 1)
    rowpar = (row & 1).astype(jnp.float32)

    B = s_ref.shape[0]
    planes = []
    s0 = []
    for b in range(B):
        sc = s_ref[b]
        x1, y1, x2, y2 = x1_ref[b], y1_ref[b], x2_ref[b], y2_ref[b]
        valid = (sc > _SCORE_THR) & (x2 > x1) & (y2 > y1)
        s0.append(jnp.where(valid, sc, neg_inf))
        area = jnp.maximum(x2 - x1, 0.0) * jnp.maximum(y2 - y1, 0.0)
        planes.append((x1, y1, x2, y2, area))

    jhalf = 63 - jax.lax.shift_right_logical(lane1, 1)
    even = (lane1 & 1) == 0

    def body(k, ss):
        keys = []
        for b in range(B):
            colmax = jnp.max(ss[b], axis=0, keepdims=True)
            bits = jax.lax.bitcast_convert_type(colmax, jnp.int32)
            packed = ((bits - _OFF) << 6) | jhalf
            alive_col = colmax > neg_inf
            keys.append(jnp.where(alive_col & even, packed, 0))
            keys.append(jnp.where(alive_col & ~even, packed, 0))
        kmaxf = jnp.max(jax.lax.bitcast_convert_type(
            jnp.concatenate(keys, axis=0), jnp.float32),
                        axis=1, keepdims=True)
        kmax4 = jax.lax.bitcast_convert_type(
            jax.lax.broadcast_in_dim(kmaxf, (2 * B, _L), (0, 1)),
            jnp.int32)

        ms, onehots, rows6, alives = [], [], [], []
        for b in range(B):
            kE = jax.lax.slice(kmax4, (2 * b, 0), (2 * b + 1, _L))
            kO = jax.lax.slice(kmax4, (2 * b + 1, 0), (2 * b + 2, _L))
            sE = jax.lax.shift_right_logical(kE, 6)
            sO = jax.lax.shift_right_logical(kO, 6)
            jE = 63 - (kE & 63)
            jO = 63 - (kO & 63)
            useE = (sE > sO) | ((sE == sO) & (jE <= jO))
            mbits = jnp.where(useE, sE, sO) + _OFF
            c = jnp.where(useE, 2 * jE, 2 * jO + 1)
            alives.append(jax.lax.slice(kE | kO, (0, 0), (1, 1)))
            m = jax.lax.bitcast_convert_type(mbits, jnp.float32)
            mask2 = (ss[b] == m) & (lane1 == c)
            rowsel = jnp.where(mask2, row, _R)
            rmin = jnp.min(rowsel, axis=0, keepdims=True)
            onehot = mask2 & (row == rmin)
            x1, y1, x2, y2, area = planes[b]
            for p in (x1, y1, x2, y2, area, rowpar):
                rows6.append(jnp.sum(jnp.where(onehot, p, 0.0),
                                     axis=0, keepdims=True))
            ms.append(m)
            onehots.append(onehot)
        ext = jnp.sum(jnp.concatenate(rows6, axis=0),
                      axis=1, keepdims=True)

        out = []
        for b in range(B):
            e = jax.lax.slice(ext, (6 * b, 0), (6 * b + 6, 1))
            bx1 = jax.lax.slice(e, (0, 0), (1, 1))
            by1 = jax.lax.slice(e, (1, 0), (2, 1))
            bx2 = jax.lax.slice(e, (2, 0), (3, 1))
            by2 = jax.lax.slice(e, (3, 0), (4, 1))
            barea = jax.lax.slice(e, (4, 0), (5, 1))
            bcls = jax.lax.slice(e, (5, 0), (6, 1))
            x1, y1, x2, y2, area = planes[b]

            iw = jnp.maximum(jnp.minimum(x2, bx2) - jnp.maximum(x1, bx1), 0.0)
            ih = jnp.maximum(jnp.minimum(y2, by2) - jnp.maximum(y1, by1), 0.0)
            inter = iw * ih
            union = jnp.maximum(area + barea - inter, 1e-9)
            kill = (inter / union > _NMS_THR) | onehots[b]

            alive_v = alives[b] != 0
            bo_ref[b, pl.ds(k, 1), 0:1] = bx1
            bo_ref[b, pl.ds(k, 1), 1:2] = by1
            bo_ref[b, pl.ds(k, 1), 2:3] = bx2
            bo_ref[b, pl.ds(k, 1), 3:4] = by2
            m11 = jax.lax.slice(ms[b], (0, 0), (1, 1))
            so_ref[b, pl.ds(k, 1), :] = jnp.where(alive_v, m11, 0.0)
            co_ref[b, pl.ds(k, 1), :] = jnp.where(alive_v,
                                                  bcls.astype(jnp.int32), -1)
            out.append(jnp.where(kill, neg_inf, ss[b]))
        return tuple(out)

    jax.lax.fori_loop(0, _MAX_PRED, body, tuple(s0))


def _to_plane(a, pad):
    B = a.shape[0]
    return jnp.pad(a, ((0, 0), (0, pad))).reshape(B, _L, _R).transpose(0, 2, 1)


@jax.jit
def kernel(boxes, scores):
    B, N, C = scores.shape[0], scores.shape[1], scores.shape[2] - 1
    M = N * C
    pad = _M - M

    bf = boxes.reshape(B, M, 4)
    x1 = _to_plane(bf[:, :, 0], pad)
    y1 = _to_plane(bf[:, :, 1], pad)
    x2 = _to_plane(bf[:, :, 2], pad)
    y2 = _to_plane(bf[:, :, 3], pad)
    sf = jnp.pad(scores[:, :, 1:].reshape(B, M), ((0, 0), (0, pad)),
                 constant_values=-1.0).reshape(B, _L, _R).transpose(0, 2, 1)

    bo, so, co = pl.pallas_call(
        _nms_kernel,
        out_shape=[
            jax.ShapeDtypeStruct((B, _MAX_PRED, 4), jnp.float32),
            jax.ShapeDtypeStruct((B, _MAX_PRED, 1), jnp.float32),
            jax.ShapeDtypeStruct((B, _MAX_PRED, 1), jnp.int32),
        ],
    )(x1, y1, x2, y2, sf)

    return bo, so[:, :, 0], co[:, :, 0]
